# baseline (speedup 1.0000x reference)
import jax
import jax.numpy as jnp
from jax import lax
from jax.experimental import pallas as pl
from jax.experimental.pallas import tpu as pltpu


def _pool2x2(ref, H, W):
    a = jnp.maximum(ref[0::2, :], ref[1::2, :])
    a4 = a.reshape(H // 2, 2, W // 2, ref.shape[-1])
    c = jnp.max(a4, axis=1)
    return c.reshape(-1, ref.shape[-1])


def _fused_kernel(x_ref, wpg_ref, bpg_ref, wfold_ref, wtheta_ref,
                  btheta_ref, bfoldT_ref, o_ref, phi_ref, g_ref, *, H, W, D):
    x = x_ref[...]
    x_bf = x.astype(jnp.bfloat16)
    pg = lax.dot_general(
        x_bf, wpg_ref[...], (((0,), (0,)), ((), ())),
        preferred_element_type=jnp.float32)
    phi_ref[...] = pg[:, :D]
    g_ref[...] = pg[:, D:]
    bpg = bpg_ref[...]
    phi = _pool2x2(phi_ref, H, W) + bpg[:, :D]
    g = _pool2x2(g_ref, H, W) + bpg[:, D:]
    m = lax.dot_general(
        phi, g, (((0,), (0,)), ((), ())),
        preferred_element_type=jnp.float32)
    wb = jnp.dot(m, wfold_ref[...],
                 preferred_element_type=jnp.float32)
    wcT = lax.dot_general(
        wb, wtheta_ref[...], (((0,), (1,)), ((), ())),
        preferred_element_type=jnp.float32)
    bcT = lax.dot_general(
        wb, btheta_ref[...], (((0,), (1,)), ((), ())),
        preferred_element_type=jnp.float32)
    o_ref[...] = (
        jnp.dot(wcT.astype(jnp.bfloat16), x_bf,
                preferred_element_type=jnp.float32)
        + bcT + bfoldT_ref[...] + x
    ).astype(o_ref.dtype)


@jax.jit
def kernel(x, theta_w, theta_b, phi_w, phi_b, g_w, g_b, W_w, W_b,
           bn_gamma, bn_beta, bn_mean, bn_var):
    B, C, H, W = x.shape
    D = theta_w.shape[1]
    N = H * W
    Nkv = (H // 2) * (W // 2)

    x3 = x.reshape(B, C, N)
    w_pg = jnp.concatenate([phi_w, g_w], axis=1).astype(jnp.bfloat16)
    b_pg = jnp.concatenate([phi_b, g_b])[None, :]

    eps = 1e-5
    scale = bn_gamma / jnp.sqrt(bn_var + eps)
    w_fold_s = (W_w * scale[None, :]) * (1.0 / Nkv)
    b_fold = (W_b - bn_mean) * scale + bn_beta

    import functools
    z = pl.pallas_call(
        functools.partial(_fused_kernel, H=H, W=W, D=D),
        out_shape=jax.ShapeDtypeStruct((B, C, N), x.dtype),
        grid=(B,),
        in_specs=[
            pl.BlockSpec((None, C, N), lambda b: (b, 0, 0)),
            pl.BlockSpec((C, 2 * D), lambda b: (0, 0)),
            pl.BlockSpec((1, 2 * D), lambda b: (0, 0)),
            pl.BlockSpec((D, C), lambda b: (0, 0)),
            pl.BlockSpec((C, D), lambda b: (0, 0)),
            pl.BlockSpec((1, D), lambda b: (0, 0)),
            pl.BlockSpec((C, 1), lambda b: (0, 0)),
        ],
        out_specs=pl.BlockSpec((None, C, N), lambda b: (b, 0, 0)),
        scratch_shapes=[pltpu.VMEM((N, D), jnp.float32),
                        pltpu.VMEM((N, D), jnp.float32)],
        compiler_params=pltpu.CompilerParams(
            dimension_semantics=("parallel",)),
    )(x3, w_pg, b_pg, w_fold_s, theta_w, theta_b[None, :], b_fold[:, None])
    return z.reshape(B, C, H, W)


# speedup vs baseline: 1.0740x; 1.0740x over previous
import functools

import jax
import jax.numpy as jnp
from jax import lax
from jax.experimental import pallas as pl
from jax.experimental.pallas import tpu as pltpu


def _pool2x2(ref, H, W):
    a = jnp.maximum(ref[0::2, :], ref[1::2, :])
    a4 = a.reshape(H // 2, 2, W // 2, a.shape[-1])
    c = jnp.max(a4, axis=1)
    return c.reshape(-1, a.shape[-1])


def _one_batch(x, wpg, bpg, wfold, wtheta, btheta, bfoldT, phi_ref, g_ref,
               H, W, D):
    x_bf = x.astype(jnp.bfloat16)
    pg = lax.dot_general(
        x_bf, wpg, (((0,), (0,)), ((), ())),
        preferred_element_type=jnp.float32)
    phi_ref[...] = pg[:, :D]
    g_ref[...] = pg[:, D:]
    phi = _pool2x2(phi_ref, H, W) + bpg[:, :D]
    g = _pool2x2(g_ref, H, W) + bpg[:, D:]
    m = lax.dot_general(
        phi, g, (((0,), (0,)), ((), ())),
        preferred_element_type=jnp.float32)
    wb = jnp.dot(m, wfold, preferred_element_type=jnp.float32)
    wcT = lax.dot_general(
        wb, wtheta, (((0,), (1,)), ((), ())),
        preferred_element_type=jnp.float32)
    bcT = lax.dot_general(
        wb, btheta, (((0,), (1,)), ((), ())),
        preferred_element_type=jnp.float32)
    return (
        jnp.dot(wcT.astype(jnp.bfloat16), x_bf,
                preferred_element_type=jnp.float32)
        + (bcT + bfoldT) + x
    )


def _fused_kernel(x_ref, wpg_ref, bpg_ref, wfold_ref, wtheta_ref,
                  btheta_ref, bfoldT_ref, o_ref, phi_ref, g_ref, *, G, H, W, D):
    wpg = wpg_ref[...]
    bpg = bpg_ref[...]
    wfold = wfold_ref[...]
    wtheta = wtheta_ref[...]
    btheta = btheta_ref[...]
    bfoldT = bfoldT_ref[...]
    for gi in range(G):
        o_ref[gi] = _one_batch(
            x_ref[gi], wpg, bpg, wfold, wtheta, btheta, bfoldT,
            phi_ref.at[gi], g_ref.at[gi], H, W, D).astype(o_ref.dtype)


@jax.jit
def kernel(x, theta_w, theta_b, phi_w, phi_b, g_w, g_b, W_w, W_b,
           bn_gamma, bn_beta, bn_mean, bn_var):
    B, C, H, W = x.shape
    D = theta_w.shape[1]
    N = H * W
    Nkv = (H // 2) * (W // 2)
    G = 4 if B % 4 == 0 else (2 if B % 2 == 0 else 1)

    x3 = x.reshape(B, C, N)
    w_pg = jnp.concatenate([phi_w, g_w], axis=1).astype(jnp.bfloat16)
    b_pg = jnp.concatenate([phi_b, g_b])[None, :]

    eps = 1e-5
    scale = bn_gamma / jnp.sqrt(bn_var + eps)
    w_fold_s = (W_w * scale[None, :]) * (1.0 / Nkv)
    b_fold = (W_b - bn_mean) * scale + bn_beta

    z = pl.pallas_call(
        functools.partial(_fused_kernel, G=G, H=H, W=W, D=D),
        out_shape=jax.ShapeDtypeStruct((B, C, N), x.dtype),
        grid=(B // G,),
        in_specs=[
            pl.BlockSpec((G, C, N), lambda b: (b, 0, 0)),
            pl.BlockSpec((C, 2 * D), lambda b: (0, 0)),
            pl.BlockSpec((1, 2 * D), lambda b: (0, 0)),
            pl.BlockSpec((D, C), lambda b: (0, 0)),
            pl.BlockSpec((C, D), lambda b: (0, 0)),
            pl.BlockSpec((1, D), lambda b: (0, 0)),
            pl.BlockSpec((C, 1), lambda b: (0, 0)),
        ],
        out_specs=pl.BlockSpec((G, C, N), lambda b: (b, 0, 0)),
        scratch_shapes=[pltpu.VMEM((G, N, D), jnp.float32),
                        pltpu.VMEM((G, N, D), jnp.float32)],
        compiler_params=pltpu.CompilerParams(
            dimension_semantics=("parallel",)),
    )(x3, w_pg, b_pg, w_fold_s, theta_w, theta_b[None, :], b_fold[:, None])
    return z.reshape(B, C, H, W)


# speedup vs baseline: 2.6499x; 2.4673x over previous
import functools

import jax
import jax.numpy as jnp
from jax import lax
from jax.experimental import pallas as pl
from jax.experimental.pallas import tpu as pltpu


def _pool2x2(ref, H, W):
    a = jnp.maximum(ref[0::2, :], ref[1::2, :])
    a4 = a.reshape(H // 2, 2, W // 2, a.shape[-1])
    c = jnp.max(a4, axis=1)
    return c.reshape(-1, a.shape[-1])


def _one_batch(x, wpg, bpg, wfold, wtheta, btheta, bfold, phi_ref, g_ref,
               H, W, D):
    x_bf = x.astype(jnp.bfloat16)
    pg = jnp.dot(x_bf, wpg, preferred_element_type=jnp.float32)
    phi_ref[...] = pg[:, :D]
    g_ref[...] = pg[:, D:]
    phi = _pool2x2(phi_ref, H, W) + bpg[:, :D]
    g = _pool2x2(g_ref, H, W) + bpg[:, D:]
    m = lax.dot_general(
        phi, g, (((0,), (0,)), ((), ())),
        preferred_element_type=jnp.float32)
    wb = jnp.dot(m, wfold, preferred_element_type=jnp.float32)
    wc = jnp.dot(wtheta, wb, preferred_element_type=jnp.float32)
    bc = jnp.dot(btheta, wb, preferred_element_type=jnp.float32) + bfold
    return (
        jnp.dot(x_bf, wc.astype(jnp.bfloat16),
                preferred_element_type=jnp.float32)
        + bc + x
    )


def _fused_kernel(x_ref, wpg_ref, bpg_ref, wfold_ref, wtheta_ref,
                  btheta_ref, bfold_ref, o_ref, phi_ref, g_ref, *, G, H, W, D):
    wpg = wpg_ref[...]
    bpg = bpg_ref[...]
    wfold = wfold_ref[...]
    wtheta = wtheta_ref[...]
    btheta = btheta_ref[...]
    bfold = bfold_ref[...]
    for gi in range(G):
        o_ref[gi] = _one_batch(
            x_ref[gi], wpg, bpg, wfold, wtheta, btheta, bfold,
            phi_ref.at[gi], g_ref.at[gi], H, W, D).astype(o_ref.dtype)


@jax.jit
def kernel(x, theta_w, theta_b, phi_w, phi_b, g_w, g_b, W_w, W_b,
           bn_gamma, bn_beta, bn_mean, bn_var):
    B, C, H, W = x.shape
    D = theta_w.shape[1]
    N = H * W
    Nkv = (H // 2) * (W // 2)
    G = 4 if B % 4 == 0 else (2 if B % 2 == 0 else 1)

    x_tok = jnp.transpose(x, (0, 2, 3, 1)).reshape(B, N, C)
    w_pg = jnp.concatenate([phi_w, g_w], axis=1).astype(jnp.bfloat16)
    b_pg = jnp.concatenate([phi_b, g_b])[None, :]

    eps = 1e-5
    scale = bn_gamma / jnp.sqrt(bn_var + eps)
    w_fold_s = (W_w * scale[None, :]) * (1.0 / Nkv)
    b_fold = (W_b - bn_mean) * scale + bn_beta

    z = pl.pallas_call(
        functools.partial(_fused_kernel, G=G, H=H, W=W, D=D),
        out_shape=jax.ShapeDtypeStruct((B, N, C), x.dtype),
        grid=(B // G,),
        in_specs=[
            pl.BlockSpec((G, N, C), lambda b: (b, 0, 0)),
            pl.BlockSpec((C, 2 * D), lambda b: (0, 0)),
            pl.BlockSpec((1, 2 * D), lambda b: (0, 0)),
            pl.BlockSpec((D, C), lambda b: (0, 0)),
            pl.BlockSpec((C, D), lambda b: (0, 0)),
            pl.BlockSpec((1, D), lambda b: (0, 0)),
            pl.BlockSpec((1, C), lambda b: (0, 0)),
        ],
        out_specs=pl.BlockSpec((G, N, C), lambda b: (b, 0, 0)),
        scratch_shapes=[pltpu.VMEM((G, N, D), jnp.float32),
                        pltpu.VMEM((G, N, D), jnp.float32)],
        compiler_params=pltpu.CompilerParams(
            dimension_semantics=("parallel",)),
    )(x_tok, w_pg, b_pg, w_fold_s, theta_w, theta_b[None, :], b_fold[None, :])
    return jnp.transpose(z.reshape(B, H, W, C), (0, 3, 1, 2))


# speedup vs baseline: 3.0665x; 1.1572x over previous
import functools

import jax
import jax.numpy as jnp
from jax import lax
from jax.experimental import pallas as pl
from jax.experimental.pallas import tpu as pltpu


def _pool2x2(ref, H, W):
    a = jnp.maximum(ref[0::2, :], ref[1::2, :])
    a4 = a.reshape(H // 2, 2, W // 2, a.shape[-1])
    c = jnp.max(a4, axis=1)
    return c.reshape(-1, a.shape[-1])


def _one_batch(x, wpg, bphi, bg, wfold, wtheta, btheta, bfold, phi_ref, g_ref,
               H, W, D):
    x_bf = x.astype(jnp.bfloat16)
    pg = jnp.dot(x_bf, wpg, preferred_element_type=jnp.float32)
    phi_ref[...] = pg[:, :D]
    g_ref[...] = pg[:, D:]
    phi = _pool2x2(phi_ref, H, W) + bphi
    g = _pool2x2(g_ref, H, W) + bg
    m = lax.dot_general(
        phi, g, (((0,), (0,)), ((), ())),
        preferred_element_type=jnp.float32)
    wb = jnp.dot(m, wfold, preferred_element_type=jnp.float32)
    wc = jnp.dot(wtheta, wb, preferred_element_type=jnp.float32)
    bc = jnp.dot(btheta, wb, preferred_element_type=jnp.float32) + bfold
    return (
        jnp.dot(x_bf, wc.astype(jnp.bfloat16),
                preferred_element_type=jnp.float32)
        + bc + x
    )


def _fused_kernel(x_ref, phiw_ref, gw_ref, thetaw_ref, ww_ref, thetab_ref,
                  phib_ref, gb_ref, wb_ref, gamma_ref, beta_ref, mean_ref,
                  var_ref, o_ref, phi_ref, g_ref, wpg_ref,
                  *, G, H, W, D, inv_nkv):
    wpg_ref[:, :D] = phiw_ref[...].astype(jnp.bfloat16)
    wpg_ref[:, D:] = gw_ref[...].astype(jnp.bfloat16)
    scale = gamma_ref[...] * lax.rsqrt(var_ref[...] + 1e-5)
    wfold = ww_ref[...] * (scale * inv_nkv)
    bfold = (wb_ref[...] - mean_ref[...]) * scale + beta_ref[...]
    wpg = wpg_ref[...]
    wtheta = thetaw_ref[...]
    btheta = thetab_ref[...]
    bphi = phib_ref[...]
    bg = gb_ref[...]
    for gi in range(G):
        o_ref[gi] = _one_batch(
            x_ref[gi], wpg, bphi, bg, wfold, wtheta, btheta, bfold,
            phi_ref.at[gi], g_ref.at[gi], H, W, D).astype(o_ref.dtype)


@jax.jit
def kernel(x, theta_w, theta_b, phi_w, phi_b, g_w, g_b, W_w, W_b,
           bn_gamma, bn_beta, bn_mean, bn_var):
    B, C, H, W = x.shape
    D = theta_w.shape[1]
    N = H * W
    Nkv = (H // 2) * (W // 2)
    G = 4 if B % 4 == 0 else (2 if B % 2 == 0 else 1)

    x_tok = jnp.transpose(x, (0, 2, 3, 1)).reshape(B, N, C)

    row = lambda v: v[None, :]
    full = lambda r, c: pl.BlockSpec((r, c), lambda b: (0, 0))
    z = pl.pallas_call(
        functools.partial(_fused_kernel, G=G, H=H, W=W, D=D,
                          inv_nkv=1.0 / Nkv),
        out_shape=jax.ShapeDtypeStruct((B, N, C), x.dtype),
        grid=(B // G,),
        in_specs=[
            pl.BlockSpec((G, N, C), lambda b: (b, 0, 0)),
            full(C, D), full(C, D), full(C, D), full(D, C),
            full(1, D), full(1, D), full(1, D),
            full(1, C), full(1, C), full(1, C), full(1, C), full(1, C),
        ],
        out_specs=pl.BlockSpec((G, N, C), lambda b: (b, 0, 0)),
        scratch_shapes=[pltpu.VMEM((G, N, D), jnp.float32),
                        pltpu.VMEM((G, N, D), jnp.float32),
                        pltpu.VMEM((C, 2 * D), jnp.bfloat16)],
        compiler_params=pltpu.CompilerParams(
            dimension_semantics=("parallel",)),
    )(x_tok, phi_w, g_w, theta_w, W_w, row(theta_b), row(phi_b), row(g_b),
      row(W_b), row(bn_gamma), row(bn_beta), row(bn_mean), row(bn_var))
    return jnp.transpose(z.reshape(B, H, W, C), (0, 3, 1, 2))


# speedup vs baseline: 3.1792x; 1.0368x over previous
import functools

import jax
import jax.numpy as jnp
from jax import lax
from jax.experimental import pallas as pl
from jax.experimental.pallas import tpu as pltpu


def _pool2x2(ref, H, W):
    a = jnp.maximum(ref[0::2, :], ref[1::2, :])
    a4 = a.reshape(H // 2, 2, W // 2, a.shape[-1])
    c = jnp.max(a4, axis=1)
    return c.reshape(-1, a.shape[-1])


def _one_batch(x, wpg, bphi, bg, wfold, wtheta, btheta, bfold, phi_ref, g_ref,
               H, W, D):
    x_bf = x.astype(jnp.bfloat16)
    pg = jnp.dot(x_bf, wpg, preferred_element_type=jnp.float32)
    phi_ref[...] = pg[:, :D]
    g_ref[...] = pg[:, D:]
    phi = _pool2x2(phi_ref, H, W) + bphi
    g = _pool2x2(g_ref, H, W) + bg
    m = lax.dot_general(
        phi, g, (((0,), (0,)), ((), ())),
        preferred_element_type=jnp.float32)
    wb = jnp.dot(m, wfold, preferred_element_type=jnp.float32)
    wc = jnp.dot(wtheta, wb, preferred_element_type=jnp.float32)
    bc = jnp.dot(btheta, wb, preferred_element_type=jnp.float32) + bfold
    return (
        jnp.dot(x_bf, wc.astype(jnp.bfloat16),
                preferred_element_type=jnp.float32)
        + bc + x
    )


def _fused_kernel(x_ref, phiw_ref, gw_ref, thetaw_ref, ww_ref, thetab_ref,
                  phib_ref, gb_ref, wb_ref, gamma_ref, beta_ref, mean_ref,
                  var_ref, o_ref, phi_ref, g_ref, wpg_ref,
                  *, G, H, W, D, inv_nkv):
    wpg_ref[:, :D] = phiw_ref[...].astype(jnp.bfloat16)
    wpg_ref[:, D:] = gw_ref[...].astype(jnp.bfloat16)
    scale = gamma_ref[...] * lax.rsqrt(var_ref[...] + 1e-5)
    wfold = ww_ref[...] * (scale * inv_nkv)
    bfold = (wb_ref[...] - mean_ref[...]) * scale + beta_ref[...]
    wpg = wpg_ref[...]
    wtheta = thetaw_ref[...]
    btheta = thetab_ref[...]
    bphi = phib_ref[...]
    bg = gb_ref[...]
    for gi in range(G):
        o_ref[gi] = _one_batch(
            x_ref[gi], wpg, bphi, bg, wfold, wtheta, btheta, bfold,
            phi_ref.at[gi], g_ref.at[gi], H, W, D).astype(o_ref.dtype)


@jax.jit
def kernel(x, theta_w, theta_b, phi_w, phi_b, g_w, g_b, W_w, W_b,
           bn_gamma, bn_beta, bn_mean, bn_var):
    B, C, H, W = x.shape
    D = theta_w.shape[1]
    N = H * W
    Nkv = (H // 2) * (W // 2)
    G = 8 if B % 8 == 0 else (4 if B % 4 == 0 else (2 if B % 2 == 0 else 1))

    x_tok = jnp.transpose(x, (0, 2, 3, 1)).reshape(B, N, C)

    row = lambda v: v[None, :]
    full = lambda r, c: pl.BlockSpec((r, c), lambda b: (0, 0))
    z = pl.pallas_call(
        functools.partial(_fused_kernel, G=G, H=H, W=W, D=D,
                          inv_nkv=1.0 / Nkv),
        out_shape=jax.ShapeDtypeStruct((B, N, C), x.dtype),
        grid=(B // G,),
        in_specs=[
            pl.BlockSpec((G, N, C), lambda b: (b, 0, 0)),
            full(C, D), full(C, D), full(C, D), full(D, C),
            full(1, D), full(1, D), full(1, D),
            full(1, C), full(1, C), full(1, C), full(1, C), full(1, C),
        ],
        out_specs=pl.BlockSpec((G, N, C), lambda b: (b, 0, 0)),
        scratch_shapes=[pltpu.VMEM((G, N, D), jnp.float32),
                        pltpu.VMEM((G, N, D), jnp.float32),
                        pltpu.VMEM((C, 2 * D), jnp.bfloat16)],
        compiler_params=pltpu.CompilerParams(
            dimension_semantics=("parallel",)),
    )(x_tok, phi_w, g_w, theta_w, W_w, row(theta_b), row(phi_b), row(g_b),
      row(W_b), row(bn_gamma), row(bn_beta), row(bn_mean), row(bn_var))
    return jnp.transpose(z.reshape(B, H, W, C), (0, 3, 1, 2))
